# baseline (speedup 1.0000x reference)
import functools

import jax
import jax.numpy as jnp
from jax import lax
from jax.experimental import pallas as pl
from jax.experimental.pallas import tpu as pltpu
from jax.experimental.pallas import tpu_sc as plsc

N = 10000
E = 320000
D = 128
H = 32

NC = 2
NS = 16
CHUNK = 125
NCHUNK = 80
EDGES_PER_TILE = CHUNK * NCHUNK
ROWS_PER_TILE = N // NS
SCHUNK = 100
SNCHUNK = 102
PAD_PER_TILE = SCHUNK * SNCHUNK - EDGES_PER_TILE
NBUF = 3
MBUF = 6

_MESH = plsc.VectorSubcoreMesh(core_axis_name="c", subcore_axis_name="s")


@functools.partial(
    pl.kernel,
    out_type=jax.ShapeDtypeStruct((NC, NS, ROWS_PER_TILE, D), jnp.float32),
    mesh=_MESH,
    scratch_types=[
        pltpu.VMEM((NCHUNK, CHUNK), jnp.int32),
        pltpu.VMEM((CHUNK, D), jnp.float32),
        pltpu.VMEM_SHARED((N, D), jnp.float32),
    ],
)
def _deg_kernel(dst_hbm, ones_hbm, z_hbm, deg_out, dst_v, ones_v, deg_sh):
    c = lax.axis_index("c")
    s = lax.axis_index("s")
    pltpu.sync_copy(dst_hbm.at[c, s], dst_v)
    pltpu.sync_copy(ones_hbm, ones_v)
    pltpu.sync_copy(z_hbm, deg_sh.at[pl.ds(s * ROWS_PER_TILE, ROWS_PER_TILE)])
    plsc.subcore_barrier()

    def body(k, carry):
        pltpu.sync_copy(ones_v, deg_sh.at[dst_v.at[k]], add=True)
        return carry

    lax.fori_loop(0, NCHUNK, body, 0)
    plsc.subcore_barrier()
    sl = pl.ds(s * ROWS_PER_TILE, ROWS_PER_TILE)
    pltpu.sync_copy(deg_sh.at[sl], deg_out.at[c, s])


@functools.partial(
    pl.kernel,
    out_type=jax.ShapeDtypeStruct((NC, NS, ROWS_PER_TILE, D), jnp.float32),
    mesh=_MESH,
    scratch_types=[
        pltpu.VMEM((MBUF, 2, SCHUNK), jnp.int32),
        pltpu.VMEM((NBUF, SCHUNK, D), jnp.float32),
        pltpu.VMEM_SHARED((N + 8, D), jnp.float32),
        [pltpu.SemaphoreType.DMA] * MBUF,
        [pltpu.SemaphoreType.DMA] * NBUF,
    ],
)
def _scat_kernel(ei_hbm, y_hbm, z_hbm, acc_out,
                 idx_v, rows_v, acc_sh, isems, gsems):
    c = lax.axis_index("c")
    s = lax.axis_index("s")
    pltpu.sync_copy(z_hbm, acc_sh.at[pl.ds(s * ROWS_PER_TILE, ROWS_PER_TILE)])

    @pl.when(s == 0)
    def _():
        pltpu.sync_copy(z_hbm.at[pl.ds(0, 8)], acc_sh.at[pl.ds(N, 8)])

    plsc.subcore_barrier()

    for j in range(MBUF):
        pltpu.async_copy(ei_hbm.at[c, s, j], idx_v.at[j], isems[j])
    for b in range(NBUF):
        pltpu.make_async_copy(ei_hbm.at[c, s, b], idx_v.at[b], isems[b]).wait()
        pltpu.async_copy(y_hbm.at[idx_v.at[b, 0]], rows_v.at[b], gsems[b])

    def body(g, carry):
        for j in range(MBUF):
            k = g * MBUF + j
            b = j % NBUF
            pltpu.make_async_copy(
                y_hbm.at[idx_v.at[j, 0]], rows_v.at[b], gsems[b]).wait()
            pltpu.sync_copy(rows_v.at[b], acc_sh.at[idx_v.at[j, 1]], add=True)

            @pl.when(k + MBUF < SNCHUNK)
            def _():
                pltpu.async_copy(ei_hbm.at[c, s, k + MBUF], idx_v.at[j],
                                 isems[j])

            jj = (j + NBUF) % MBUF

            @pl.when(k + NBUF < SNCHUNK)
            def _():
                pltpu.make_async_copy(
                    ei_hbm.at[c, s, 0], idx_v.at[jj], isems[jj]).wait()
                pltpu.async_copy(y_hbm.at[idx_v.at[jj, 0]], rows_v.at[b],
                                 gsems[b])
        return carry

    lax.fori_loop(0, SNCHUNK // MBUF, body, 0)
    plsc.subcore_barrier()
    sl = pl.ds(s * ROWS_PER_TILE, ROWS_PER_TILE)
    pltpu.sync_copy(acc_sh.at[sl], acc_out.at[c, s])


def _mm_body(state_ref, w_ref, deg_ref, y_ref, dinv_ref):
    deg = deg_ref[0][:, 0:1] + deg_ref[1][:, 0:1] + 1.0
    dinv = lax.rsqrt(deg)
    xw = jnp.dot(state_ref[...], w_ref[...], preferred_element_type=jnp.float32)
    y_ref[...] = xw * dinv
    dinv_ref[...] = dinv


_mm_call = pl.pallas_call(
    _mm_body,
    out_shape=[
        jax.ShapeDtypeStruct((N, D), jnp.float32),
        jax.ShapeDtypeStruct((N, 1), jnp.float32),
    ],
)


def _mlp_body(acc_ref, y_ref, dinv_ref, state_ref, bg_ref,
              w1_ref, b1_ref, w2_ref, b2_ref, w3_ref, b3_ref, act_ref):
    g = (acc_ref[0] + acc_ref[1] + y_ref[...]) * dinv_ref[...] + bg_ref[...]
    g = jnp.maximum(g, 0.0) + state_ref[...]
    h = jnp.dot(g, w1_ref[...], preferred_element_type=jnp.float32) + b1_ref[...]
    h = jnp.where(h >= 0, h, 0.01 * h)
    h = jnp.dot(h, w2_ref[...], preferred_element_type=jnp.float32) + b2_ref[...]
    h = jnp.where(h >= 0, h, 0.01 * h)
    z = jnp.dot(h, w3_ref[...], preferred_element_type=jnp.float32) + b3_ref[...]
    conc = jax.nn.softplus(z)
    act_ref[...] = conc / (jnp.sum(conc) + 1e-20)


_mlp_call = pl.pallas_call(
    _mlp_body,
    out_shape=jax.ShapeDtypeStruct((N, 1), jnp.float32),
)


def kernel(state, edge_index, W_gcn, b_gcn, W1, b1, W2, b2, W3, b3,
           deterministic):
    dst4 = edge_index[1].reshape(NC, NS, NCHUNK, CHUNK)
    ei_t = edge_index.reshape(2, NC, NS, EDGES_PER_TILE)
    pad = jnp.full((2, NC, NS, PAD_PER_TILE), N, jnp.int32).at[0].set(0)
    ei5 = (jnp.concatenate([ei_t, pad], axis=-1)
           .reshape(2, NC, NS, SNCHUNK, SCHUNK).transpose(1, 2, 3, 0, 4))
    ones = jnp.ones((CHUNK, D), jnp.float32)
    zD = jnp.zeros((ROWS_PER_TILE, D), jnp.float32)
    z1 = zD

    deg_parts = _deg_kernel(dst4, ones, z1).reshape(NC, N, D)
    y, dinv = _mm_call(state, W_gcn, deg_parts)
    acc_parts = _scat_kernel(ei5, y, zD).reshape(NC, N, D)
    act = _mlp_call(acc_parts, y, dinv, state,
                    b_gcn.reshape(1, D), W1, b1.reshape(1, H),
                    W2, b2.reshape(1, H), W3, b3.reshape(1, 1))
    return act.reshape(N // 8, 8)


# speedup vs baseline: 1.8569x; 1.8569x over previous
import functools

import jax
import jax.numpy as jnp
from jax import lax
from jax.experimental import pallas as pl
from jax.experimental.pallas import tpu as pltpu
from jax.experimental.pallas import tpu_sc as plsc

N = 10000
E = 320000
D = 128
H = 32

NC = 2
NS = 16
CHUNK = 125
NCHUNK = 80
EDGES_PER_TILE = CHUNK * NCHUNK
ROWS_PER_TILE = N // NS
SCHUNK = 125
SNCHUNK = 80
NBUF = 2
MBUF = 4

_MESH = plsc.VectorSubcoreMesh(core_axis_name="c", subcore_axis_name="s")


@functools.partial(
    pl.kernel,
    out_type=jax.ShapeDtypeStruct((NC, NS, ROWS_PER_TILE, D), jnp.float32),
    mesh=_MESH,
    scratch_types=[
        pltpu.VMEM((NCHUNK, CHUNK), jnp.int32),
        pltpu.VMEM((CHUNK, D), jnp.float32),
        pltpu.VMEM_SHARED((N, D), jnp.float32),
    ],
)
def _deg_kernel(dst_hbm, ones_hbm, z_hbm, deg_out, dst_v, ones_v, deg_sh):
    c = lax.axis_index("c")
    s = lax.axis_index("s")
    pltpu.sync_copy(dst_hbm.at[c, s], dst_v)
    pltpu.sync_copy(ones_hbm, ones_v)
    pltpu.sync_copy(z_hbm, deg_sh.at[pl.ds(s * ROWS_PER_TILE, ROWS_PER_TILE)])
    plsc.subcore_barrier()

    def body(k, carry):
        pltpu.sync_copy(ones_v, deg_sh.at[dst_v.at[k]], add=True)
        return carry

    lax.fori_loop(0, NCHUNK, body, 0)
    plsc.subcore_barrier()
    sl = pl.ds(s * ROWS_PER_TILE, ROWS_PER_TILE)
    pltpu.sync_copy(deg_sh.at[sl], deg_out.at[c, s])


@functools.partial(
    pl.kernel,
    out_type=jax.ShapeDtypeStruct((NC, NS, ROWS_PER_TILE, D), jnp.float32),
    mesh=_MESH,
    scratch_types=[
        pltpu.VMEM((MBUF, 2, SCHUNK), jnp.int32),
        pltpu.VMEM((NBUF, SCHUNK, D), jnp.float32),
        pltpu.VMEM_SHARED((N, D), jnp.float32),
        [pltpu.SemaphoreType.DMA] * MBUF,
        [pltpu.SemaphoreType.DMA] * NBUF,
    ],
)
def _scat_kernel(ei_hbm, y_hbm, z_hbm, acc_out,
                 idx_v, rows_v, acc_sh, isems, gsems):
    c = lax.axis_index("c")
    s = lax.axis_index("s")
    pltpu.sync_copy(z_hbm, acc_sh.at[pl.ds(s * ROWS_PER_TILE, ROWS_PER_TILE)])
    plsc.subcore_barrier()

    for j in range(MBUF):
        pltpu.async_copy(ei_hbm.at[c, s, j], idx_v.at[j], isems[j])
    for b in range(NBUF):
        pltpu.make_async_copy(ei_hbm.at[c, s, b], idx_v.at[b], isems[b]).wait()
        pltpu.async_copy(y_hbm.at[idx_v.at[b, 0]], rows_v.at[b], gsems[b])

    def body(g, carry):
        for j in range(MBUF):
            k = g * MBUF + j
            b = j % NBUF
            pltpu.make_async_copy(
                y_hbm.at[idx_v.at[j, 0]], rows_v.at[b], gsems[b]).wait()
            pltpu.sync_copy(rows_v.at[b], acc_sh.at[idx_v.at[j, 1]], add=True)

            @pl.when(k + MBUF < SNCHUNK)
            def _():
                pltpu.async_copy(ei_hbm.at[c, s, k + MBUF], idx_v.at[j],
                                 isems[j])

            jj = (j + NBUF) % MBUF

            @pl.when(k + NBUF < SNCHUNK)
            def _():
                pltpu.make_async_copy(
                    ei_hbm.at[c, s, 0], idx_v.at[jj], isems[jj]).wait()
                pltpu.async_copy(y_hbm.at[idx_v.at[jj, 0]], rows_v.at[b],
                                 gsems[b])
        return carry

    lax.fori_loop(0, SNCHUNK // MBUF, body, 0)
    plsc.subcore_barrier()
    sl = pl.ds(s * ROWS_PER_TILE, ROWS_PER_TILE)
    pltpu.sync_copy(acc_sh.at[sl], acc_out.at[c, s])


def _xw_body(state_ref, w_ref, xw_ref):
    xw_ref[...] = jnp.dot(state_ref[...], w_ref[...],
                          preferred_element_type=jnp.float32)


_xw_call = pl.pallas_call(
    _xw_body, out_shape=jax.ShapeDtypeStruct((N, D), jnp.float32))


def _scale_body(xw_ref, deg_ref, y_ref, dinv_ref):
    deg = deg_ref[0][:, 0:1] + deg_ref[1][:, 0:1] + 1.0
    dinv = lax.rsqrt(deg)
    y_ref[...] = xw_ref[...] * dinv
    dinv_ref[...] = dinv


_scale_call = pl.pallas_call(
    _scale_body,
    out_shape=[
        jax.ShapeDtypeStruct((N, D), jnp.float32),
        jax.ShapeDtypeStruct((N, 1), jnp.float32),
    ],
)


def _mlp_body(acc_ref, y_ref, dinv_ref, state_ref, bg_ref,
              w1_ref, b1_ref, w2_ref, b2_ref, w3_ref, b3_ref, act_ref):
    g = (acc_ref[0] + acc_ref[1] + y_ref[...]) * dinv_ref[...] + bg_ref[...]
    g = jnp.maximum(g, 0.0) + state_ref[...]
    h = jnp.dot(g, w1_ref[...], preferred_element_type=jnp.float32) + b1_ref[...]
    h = jnp.where(h >= 0, h, 0.01 * h)
    h = jnp.dot(h, w2_ref[...], preferred_element_type=jnp.float32) + b2_ref[...]
    h = jnp.where(h >= 0, h, 0.01 * h)
    z = jnp.dot(h, w3_ref[...], preferred_element_type=jnp.float32) + b3_ref[...]
    conc = jax.nn.softplus(z)
    act_ref[...] = conc / (jnp.sum(conc) + 1e-20)


_mlp_call = pl.pallas_call(
    _mlp_body,
    out_shape=jax.ShapeDtypeStruct((N, 1), jnp.float32),
)


def kernel(state, edge_index, W_gcn, b_gcn, W1, b1, W2, b2, W3, b3,
           deterministic):
    dst4 = edge_index[1].reshape(NC, NS, NCHUNK, CHUNK)
    ei5 = (edge_index.reshape(2, NC, NS, SNCHUNK, SCHUNK)
           .transpose(1, 2, 3, 0, 4))
    ones = jnp.ones((CHUNK, D), jnp.float32)
    zD = jnp.zeros((ROWS_PER_TILE, D), jnp.float32)
    z1 = zD

    xw = _xw_call(state, W_gcn)
    deg_parts = _deg_kernel(dst4, ones, z1).reshape(NC, N, D)
    y, dinv = _scale_call(xw, deg_parts)
    acc_parts = _scat_kernel(ei5, y, zD).reshape(NC, N, D)
    act = _mlp_call(acc_parts, y, dinv, state,
                    b_gcn.reshape(1, D), W1, b1.reshape(1, H),
                    W2, b2.reshape(1, H), W3, b3.reshape(1, 1))
    return act.reshape(N // 8, 8)


# speedup vs baseline: 1.8668x; 1.0053x over previous
import functools

import jax
import jax.numpy as jnp
from jax import lax
from jax.experimental import pallas as pl
from jax.experimental.pallas import tpu as pltpu
from jax.experimental.pallas import tpu_sc as plsc

N = 10000
E = 320000
D = 128
H = 32

NC = 2
NS = 16
CHUNK = 125
NCHUNK = 80
EDGES_PER_TILE = CHUNK * NCHUNK
ROWS_PER_TILE = N // NS
SCHUNK = 125
SNCHUNK = 80
NBUF = 2
MBUF = 4

_MESH = plsc.VectorSubcoreMesh(core_axis_name="c", subcore_axis_name="s")


@functools.partial(
    pl.kernel,
    out_type=jax.ShapeDtypeStruct((NC, NS, ROWS_PER_TILE, D), jnp.float32),
    mesh=_MESH,
    scratch_types=[
        pltpu.VMEM((NCHUNK, CHUNK), jnp.int32),
        pltpu.VMEM((CHUNK, D), jnp.float32),
        pltpu.VMEM_SHARED((N, D), jnp.float32),
    ],
)
def _deg_kernel(dst_hbm, ones_hbm, z_hbm, deg_out, dst_v, ones_v, deg_sh):
    c = lax.axis_index("c")
    s = lax.axis_index("s")
    pltpu.sync_copy(dst_hbm.at[c, s], dst_v)
    pltpu.sync_copy(ones_hbm, ones_v)
    pltpu.sync_copy(z_hbm, deg_sh.at[pl.ds(s * ROWS_PER_TILE, ROWS_PER_TILE)])
    plsc.subcore_barrier()

    def body(k, carry):
        pltpu.sync_copy(ones_v, deg_sh.at[dst_v.at[k]], add=True)
        return carry

    lax.fori_loop(0, NCHUNK, body, 0)
    plsc.subcore_barrier()
    sl = pl.ds(s * ROWS_PER_TILE, ROWS_PER_TILE)
    pltpu.sync_copy(deg_sh.at[sl], deg_out.at[c, s])


@functools.partial(
    pl.kernel,
    out_type=jax.ShapeDtypeStruct((NC, NS, ROWS_PER_TILE, D), jnp.float32),
    mesh=_MESH,
    scratch_types=[
        pltpu.VMEM((MBUF, 2, SCHUNK), jnp.int32),
        pltpu.VMEM((NBUF, SCHUNK, D), jnp.float32),
        pltpu.VMEM_SHARED((N, D), jnp.float32),
        [pltpu.SemaphoreType.DMA] * MBUF,
        [pltpu.SemaphoreType.DMA] * NBUF,
    ],
)
def _scat_kernel(ei_hbm, y_hbm, z_hbm, acc_out,
                 idx_v, rows_v, acc_sh, isems, gsems):
    c = lax.axis_index("c")
    s = lax.axis_index("s")
    pltpu.sync_copy(z_hbm, acc_sh.at[pl.ds(s * ROWS_PER_TILE, ROWS_PER_TILE)])
    plsc.subcore_barrier()

    for j in range(MBUF):
        pltpu.async_copy(ei_hbm.at[c, s, j], idx_v.at[j], isems[j])
    for b in range(NBUF):
        pltpu.make_async_copy(ei_hbm.at[c, s, b], idx_v.at[b], isems[b]).wait()
        pltpu.async_copy(y_hbm.at[idx_v.at[b, 0]], rows_v.at[b], gsems[b])

    def body(g, carry):
        for j in range(MBUF):
            k = g * MBUF + j
            b = j % NBUF
            pltpu.make_async_copy(
                y_hbm.at[idx_v.at[j, 0]], rows_v.at[b], gsems[b]).wait()
            pltpu.sync_copy(rows_v.at[b], acc_sh.at[idx_v.at[j, 1]], add=True)

            @pl.when(k + MBUF < SNCHUNK)
            def _():
                pltpu.async_copy(ei_hbm.at[c, s, k + MBUF], idx_v.at[j],
                                 isems[j])

            jj = (j + NBUF) % MBUF

            @pl.when(k + NBUF < SNCHUNK)
            def _():
                pltpu.make_async_copy(
                    ei_hbm.at[c, s, 0], idx_v.at[jj], isems[jj]).wait()
                pltpu.async_copy(y_hbm.at[idx_v.at[jj, 0]], rows_v.at[b],
                                 gsems[b])
        return carry

    lax.fori_loop(0, SNCHUNK // MBUF, body, 0)
    plsc.subcore_barrier()
    sl = pl.ds(s * ROWS_PER_TILE, ROWS_PER_TILE)
    pltpu.sync_copy(acc_sh.at[sl], acc_out.at[c, s])


def _mm_body(state_ref, w_ref, deg_ref, y_ref, dinv_ref):
    deg = deg_ref[0][:, 0:1] + deg_ref[1][:, 0:1] + 1.0
    dinv = lax.rsqrt(deg)
    xw = jnp.dot(state_ref[...], w_ref[...], preferred_element_type=jnp.float32)
    y_ref[...] = xw * dinv
    dinv_ref[...] = dinv


_mm_call = pl.pallas_call(
    _mm_body,
    out_shape=[
        jax.ShapeDtypeStruct((N, D), jnp.float32),
        jax.ShapeDtypeStruct((N, 1), jnp.float32),
    ],
)


def _mlp_body(acc_ref, y_ref, dinv_ref, state_ref, bg_ref,
              w1_ref, b1_ref, w2_ref, b2_ref, w3_ref, b3_ref, act_ref):
    g = (acc_ref[0] + acc_ref[1] + y_ref[...]) * dinv_ref[...] + bg_ref[...]
    g = jnp.maximum(g, 0.0) + state_ref[...]
    h = jnp.dot(g, w1_ref[...], preferred_element_type=jnp.float32) + b1_ref[...]
    h = jnp.where(h >= 0, h, 0.01 * h)
    h = jnp.dot(h, w2_ref[...], preferred_element_type=jnp.float32) + b2_ref[...]
    h = jnp.where(h >= 0, h, 0.01 * h)
    z = jnp.dot(h, w3_ref[...], preferred_element_type=jnp.float32) + b3_ref[...]
    conc = jax.nn.softplus(z)
    act_ref[...] = conc / (jnp.sum(conc) + 1e-20)


_mlp_call = pl.pallas_call(
    _mlp_body,
    out_shape=jax.ShapeDtypeStruct((N, 1), jnp.float32),
)


def kernel(state, edge_index, W_gcn, b_gcn, W1, b1, W2, b2, W3, b3,
           deterministic):
    dst4 = edge_index[1].reshape(NC, NS, NCHUNK, CHUNK)
    ei5 = (edge_index.reshape(2, NC, NS, SNCHUNK, SCHUNK)
           .transpose(1, 2, 3, 0, 4))
    ones = jnp.ones((CHUNK, D), jnp.float32)
    zD = jnp.zeros((ROWS_PER_TILE, D), jnp.float32)
    z1 = zD

    deg_parts = _deg_kernel(dst4, ones, z1).reshape(NC, N, D)
    y, dinv = _mm_call(state, W_gcn, deg_parts)
    acc_parts = _scat_kernel(ei5, y, zD).reshape(NC, N, D)
    act = _mlp_call(acc_parts, y, dinv, state,
                    b_gcn.reshape(1, D), W1, b1.reshape(1, H),
                    W2, b2.reshape(1, H), W3, b3.reshape(1, 1))
    return act.reshape(N // 8, 8)
